# baseline (speedup 1.0000x reference)
import functools

import jax
import jax.numpy as jnp
from jax import lax
from jax.experimental import pallas as pl
from jax.experimental.pallas import tpu as pltpu
from jax.experimental.pallas import tpu_sc as plsc

_P_SA0 = 0.1
_P_SA1 = 0.1
_G_SA0 = 0.003
_G_SA1 = 3e-06

_M = 16384
_N = 4096
_TOT = _M * _N
_NC = 2
_NS = 16
_NW = _NC * _NS
_PER_W = _TOT // _NW
_CH = 16384
_STEPS = _PER_W // _CH
_LANES = 16


def _saf_chunk(xb, pb):
    def body(i, _):
        sl = pl.ds(i * _LANES, _LANES)
        pv = pb[sl]
        xv = xb[sl]
        ov = jnp.where(pv <= jnp.float32(_P_SA0), jnp.float32(_G_SA0), xv)
        ov = jnp.where(pv > jnp.float32(1.0 - _P_SA1), jnp.float32(_G_SA1), ov)
        xb[sl] = ov
        return 0

    lax.fori_loop(0, _CH // _LANES, body, 0)


_mesh = plsc.VectorSubcoreMesh(core_axis_name="c", subcore_axis_name="s")


@functools.partial(
    pl.kernel,
    mesh=_mesh,
    out_type=jax.ShapeDtypeStruct((_TOT,), jnp.float32),
    scratch_types=[
        pltpu.VMEM((_CH,), jnp.float32),
        pltpu.VMEM((_CH,), jnp.float32),
        pltpu.SemaphoreType.DMA,
    ],
)
def _saf_sc(x_hbm, p_hbm, o_hbm, xb, pb, sem):
    wid = lax.axis_index("s") * _NC + lax.axis_index("c")
    base = wid * _PER_W

    def step(s, _):
        off = base + s * _CH
        pltpu.sync_copy(x_hbm.at[pl.ds(off, _CH)], xb)
        pltpu.sync_copy(p_hbm.at[pl.ds(off, _CH)], pb)
        _saf_chunk(xb, pb)
        pltpu.sync_copy(xb, o_hbm.at[pl.ds(off, _CH)])
        return 0

    lax.fori_loop(0, _STEPS, step, 0)


def kernel(input, p_state):
    out = _saf_sc(input.reshape(_TOT), p_state.reshape(_TOT))
    return out.reshape(_M, _N)


# speedup vs baseline: 1.0146x; 1.0146x over previous
import functools

import jax
import jax.numpy as jnp
from jax import lax
from jax.experimental import pallas as pl
from jax.experimental.pallas import tpu as pltpu
from jax.experimental.pallas import tpu_sc as plsc

_P_SA0 = 0.1
_P_SA1 = 0.1
_G_SA0 = 0.003
_G_SA1 = 3e-06

_M = 16384
_N = 4096
_TOT = _M * _N
_NC = 2
_NS = 16
_NW = _NC * _NS
_PER_W = _TOT // _NW
_CH = 16384
_STEPS = _PER_W // _CH
_LANES = 16
_UNROLL = 8


def _saf_chunk(xb, pb, ob):
    def body(i, _):
        base = i * (_LANES * _UNROLL)
        for u in range(_UNROLL):
            sl = pl.ds(base + u * _LANES, _LANES)
            pv = pb[sl]
            xv = xb[sl]
            ov = jnp.where(pv <= jnp.float32(_P_SA0), jnp.float32(_G_SA0), xv)
            ov = jnp.where(pv > jnp.float32(1.0 - _P_SA1), jnp.float32(_G_SA1), ov)
            ob[sl] = ov
        return 0

    lax.fori_loop(0, _CH // (_LANES * _UNROLL), body, 0)


_mesh = plsc.VectorSubcoreMesh(core_axis_name="c", subcore_axis_name="s")


@functools.partial(
    pl.kernel,
    mesh=_mesh,
    out_type=jax.ShapeDtypeStruct((_TOT,), jnp.float32),
    scratch_types=[
        pltpu.VMEM((2, _CH), jnp.float32),
        pltpu.VMEM((2, _CH), jnp.float32),
        pltpu.VMEM((2, _CH), jnp.float32),
        pltpu.SemaphoreType.DMA((2,)),
        pltpu.SemaphoreType.DMA((2,)),
        pltpu.SemaphoreType.DMA((2,)),
    ],
)
def _saf_sc(x_hbm, p_hbm, o_hbm, xb, pb, ob, lx_sem, lp_sem, st_sem):
    wid = lax.axis_index("s") * _NC + lax.axis_index("c")
    base = wid * _PER_W

    def load(s, b):
        off = base + s * _CH
        pltpu.make_async_copy(x_hbm.at[pl.ds(off, _CH)], xb.at[b], lx_sem.at[b]).start()
        pltpu.make_async_copy(p_hbm.at[pl.ds(off, _CH)], pb.at[b], lp_sem.at[b]).start()

    load(0, 0)
    load(1, 1)

    def step(s, _):
        b = lax.rem(s, 2)
        off = base + s * _CH
        pltpu.make_async_copy(x_hbm.at[pl.ds(off, _CH)], xb.at[b], lx_sem.at[b]).wait()
        pltpu.make_async_copy(p_hbm.at[pl.ds(off, _CH)], pb.at[b], lp_sem.at[b]).wait()

        @pl.when(s >= 2)
        def _():
            prev = base + (s - 2) * _CH
            pltpu.make_async_copy(ob.at[b], o_hbm.at[pl.ds(prev, _CH)], st_sem.at[b]).wait()

        _saf_chunk(xb.at[b], pb.at[b], ob.at[b])
        pltpu.make_async_copy(ob.at[b], o_hbm.at[pl.ds(off, _CH)], st_sem.at[b]).start()

        @pl.when(s + 2 < _STEPS)
        def _():
            load(s + 2, b)

        return 0

    lax.fori_loop(0, _STEPS, step, 0)

    def drain(b, _):
        off = base + (_STEPS - 2 + b) * _CH
        pltpu.make_async_copy(ob.at[b], o_hbm.at[pl.ds(off, _CH)],
                              st_sem.at[lax.rem(_STEPS - 2 + b, 2)]).wait()
        return 0

    lax.fori_loop(0, 2, drain, 0)


def kernel(input, p_state):
    out = _saf_sc(input.reshape(_TOT), p_state.reshape(_TOT))
    return out.reshape(_M, _N)


# speedup vs baseline: 5.7836x; 5.7002x over previous
import functools

import jax
import jax.numpy as jnp
from jax import lax
from jax.experimental import pallas as pl
from jax.experimental.pallas import tpu as pltpu
from jax.experimental.pallas import tpu_sc as plsc

_P_SA0 = 0.1
_P_SA1 = 0.1
_G_SA0 = 0.003
_G_SA1 = 3e-06

_M = 16384
_N = 4096
_NC = 2
_NS = 16
_NW = _NC * _NS
_ROWS_W = _M // _NW
_R = 4
_STEPS = _ROWS_W // _R
_LANES = 16


def _saf_chunk(xb, pb, ob):
    for r in range(_R):
        @plsc.parallel_loop(0, _N, step=_LANES, unroll=8)
        def _(c):
            sl = pl.ds(c, _LANES)
            pv = pb[r, sl]
            xv = xb[r, sl]
            ov = jnp.where(pv <= jnp.float32(_P_SA0), jnp.float32(_G_SA0), xv)
            ov = jnp.where(pv > jnp.float32(1.0 - _P_SA1), jnp.float32(_G_SA1), ov)
            ob[r, sl] = ov


_mesh = plsc.VectorSubcoreMesh(core_axis_name="c", subcore_axis_name="s")


@functools.partial(
    pl.kernel,
    mesh=_mesh,
    out_type=jax.ShapeDtypeStruct((_M, _N), jnp.float32),
    scratch_types=[
        pltpu.VMEM((2, _R, _N), jnp.float32),
        pltpu.VMEM((2, _R, _N), jnp.float32),
        pltpu.VMEM((2, _R, _N), jnp.float32),
        pltpu.SemaphoreType.DMA((2,)),
        pltpu.SemaphoreType.DMA((2,)),
        pltpu.SemaphoreType.DMA((2,)),
    ],
)
def _saf_sc(x_hbm, p_hbm, o_hbm, xb, pb, ob, lx_sem, lp_sem, st_sem):
    wid = lax.axis_index("s") * _NC + lax.axis_index("c")
    base = wid * _ROWS_W

    def load(s, b):
        row = base + s * _R
        pltpu.make_async_copy(x_hbm.at[pl.ds(row, _R)], xb.at[b], lx_sem.at[b]).start()
        pltpu.make_async_copy(p_hbm.at[pl.ds(row, _R)], pb.at[b], lp_sem.at[b]).start()

    load(0, 0)
    load(1, 1)

    def step(s, _):
        b = lax.rem(s, 2)
        row = base + s * _R
        pltpu.make_async_copy(x_hbm.at[pl.ds(row, _R)], xb.at[b], lx_sem.at[b]).wait()
        pltpu.make_async_copy(p_hbm.at[pl.ds(row, _R)], pb.at[b], lp_sem.at[b]).wait()

        @pl.when(s >= 2)
        def _():
            prev = base + (s - 2) * _R
            pltpu.make_async_copy(ob.at[b], o_hbm.at[pl.ds(prev, _R)], st_sem.at[b]).wait()

        _saf_chunk(xb.at[b], pb.at[b], ob.at[b])
        pltpu.make_async_copy(ob.at[b], o_hbm.at[pl.ds(row, _R)], st_sem.at[b]).start()

        @pl.when(s + 2 < _STEPS)
        def _():
            load(s + 2, b)

        return 0

    lax.fori_loop(0, _STEPS, step, 0)

    for k in range(2):
        s = _STEPS - 2 + k
        row = base + s * _R
        pltpu.make_async_copy(ob.at[s % 2], o_hbm.at[pl.ds(row, _R)],
                              st_sem.at[s % 2]).wait()


def kernel(input, p_state):
    return _saf_sc(input, p_state)


# speedup vs baseline: 5.7949x; 1.0019x over previous
import functools

import jax
import jax.numpy as jnp
from jax import lax
from jax.experimental import pallas as pl
from jax.experimental.pallas import tpu as pltpu
from jax.experimental.pallas import tpu_sc as plsc

_P_SA0 = 0.1
_P_SA1 = 0.1
_G_SA0 = 0.003
_G_SA1 = 3e-06

_M = 16384
_N = 4096
_NC = 2
_NS = 16
_NW = _NC * _NS
_ROWS_W = _M // _NW
_CR = 8
_CN = 2048
_CSTEPS_R = _ROWS_W // _CR
_CSTEPS_N = _N // _CN
_STEPS = _CSTEPS_R * _CSTEPS_N
_LANES = 16
_NBUF = 3


def _saf_chunk(xb, pb):
    for r in range(_CR):
        @plsc.parallel_loop(0, _CN, step=_LANES, unroll=8)
        def _(c):
            sl = pl.ds(c, _LANES)
            pv = pb[r, sl]
            xv = xb[r, sl]
            ov = jnp.where(pv <= jnp.float32(_P_SA0), jnp.float32(_G_SA0), xv)
            ov = jnp.where(pv > jnp.float32(1.0 - _P_SA1), jnp.float32(_G_SA1), ov)
            xb[r, sl] = ov


_mesh = plsc.VectorSubcoreMesh(core_axis_name="c", subcore_axis_name="s")


@functools.partial(
    pl.kernel,
    mesh=_mesh,
    out_type=jax.ShapeDtypeStruct((_M, _N), jnp.float32),
    scratch_types=[
        pltpu.VMEM((_NBUF, _CR, _CN), jnp.float32),
        pltpu.VMEM((_NBUF, _CR, _CN), jnp.float32),
        pltpu.SemaphoreType.DMA((_NBUF,)),
        pltpu.SemaphoreType.DMA((_NBUF,)),
        pltpu.SemaphoreType.DMA((_NBUF,)),
    ],
)
def _saf_sc(x_hbm, p_hbm, o_hbm, xb, pb, lx_sem, lp_sem, st_sem):
    wid = lax.axis_index("s") * _NC + lax.axis_index("c")
    base = wid * _ROWS_W

    def chunk_slice(s):
        row = base + lax.div(s, _CSTEPS_N) * _CR
        col = lax.rem(s, _CSTEPS_N) * _CN
        return (pl.ds(row, _CR), pl.ds(col, _CN))

    def load(s, b):
        sl = chunk_slice(s)
        pltpu.make_async_copy(x_hbm.at[sl[0], sl[1]], xb.at[b], lx_sem.at[b]).start()
        pltpu.make_async_copy(p_hbm.at[sl[0], sl[1]], pb.at[b], lp_sem.at[b]).start()

    def wait_store(s, b):
        sl = chunk_slice(s)
        pltpu.make_async_copy(xb.at[b], o_hbm.at[sl[0], sl[1]], st_sem.at[b]).wait()

    load(0, 0)
    load(1, 1)

    def step(s, _):
        b = lax.rem(s, _NBUF)
        sl = chunk_slice(s)
        pltpu.make_async_copy(x_hbm.at[sl[0], sl[1]], xb.at[b], lx_sem.at[b]).wait()
        pltpu.make_async_copy(p_hbm.at[sl[0], sl[1]], pb.at[b], lp_sem.at[b]).wait()

        _saf_chunk(xb.at[b], pb.at[b])
        pltpu.make_async_copy(xb.at[b], o_hbm.at[sl[0], sl[1]], st_sem.at[b]).start()

        @pl.when(s + 2 < _STEPS)
        def _():
            b2 = lax.rem(s + 2, _NBUF)

            @pl.when(s >= 1)
            def _():
                wait_store(s - 1, b2)

            load(s + 2, b2)

        return 0

    lax.fori_loop(0, _STEPS, step, 0)

    for s in range(_STEPS - 3, _STEPS):
        wait_store(s, s % _NBUF)


def kernel(input, p_state):
    return _saf_sc(input, p_state)
